# baseline (speedup 1.0000x reference)
import functools

import jax
import jax.numpy as jnp
from jax import lax
from jax.experimental import pallas as pl
from jax.experimental.pallas import tpu as pltpu
from jax.experimental.pallas import tpu_sc as plsc

N_CLS = 1000
PRE = 5
NCTX = 16
TOT = 77
SUF = TOT - PRE - NCTX
D = 512
NW = 32
CHUNK = 112

PRE_ROWS = PRE * N_CLS
CTX_ROWS = NCTX * N_CLS
SUF_ROWS = SUF * N_CLS
CTX_ROW0 = PRE_ROWS
SUF_ROW0 = PRE_ROWS + CTX_ROWS

CTX_PER_W = 504
SUF_PER_W = 1752
CTX_LENS = [CHUNK] * 4 + [CTX_PER_W - 4 * CHUNK]
SUF_LENS = [CHUNK] * 15 + [SUF_PER_W - 15 * CHUNK]

_mesh = plsc.VectorSubcoreMesh(core_axis_name="c", subcore_axis_name="s")


@functools.partial(
    pl.kernel,
    mesh=_mesh,
    out_type=jax.ShapeDtypeStruct((TOT * N_CLS, D), jnp.float32),
    scratch_types=[
        pltpu.VMEM((CHUNK,), jnp.int32),
        pltpu.VMEM((CHUNK,), jnp.int32),
        pltpu.VMEM((CHUNK, D), jnp.float32),
        pltpu.VMEM((CHUNK, D), jnp.float32),
        pltpu.SemaphoreType.DMA,
        pltpu.SemaphoreType.DMA,
        pltpu.SemaphoreType.DMA,
        pltpu.SemaphoreType.DMA,
    ],
)
def _assemble(ctx_hbm, pre_hbm, suf_hbm, out_hbm,
              ia, ib, ra, rb, ga, gb, oa, ob):
    idx = [ia, ib]
    rows = [ra, rb]
    gsem = [ga, gb]
    osem = [oa, ob]
    wid = lax.axis_index("s") * 2 + lax.axis_index("c")
    j16 = lax.iota(jnp.int32, 16)
    i32 = jnp.int32

    v_ctx = jnp.minimum(wid * CTX_PER_W, CTX_ROWS - CTX_PER_W)
    v_suf = jnp.minimum(wid * SUF_PER_W, SUF_ROWS - SUF_PER_W)

    def plane_of(v, nplanes):
        t0 = jnp.where(v >= N_CLS, 1, 0)
        for r in range(2, nplanes):
            t0 = t0 + jnp.where(v >= r * N_CLS, 1, 0)
        return t0

    k_ctx = plane_of(v_ctx, NCTX)
    t_suf = plane_of(v_suf, SUF)

    a = jnp.minimum(wid * 160, PRE_ROWS - 160)
    pcp = [
        pltpu.make_async_copy(
            pre_hbm.at[pl.ds(a + 80 * h, 80)], rows[h].at[pl.ds(0, 80)],
            gsem[h])
        for h in range(2)
    ]
    ocp = [
        pltpu.make_async_copy(
            rows[h].at[pl.ds(0, 80)], out_hbm.at[pl.ds(a + 80 * h, 80)],
            osem[h])
        for h in range(2)
    ]
    pcp[0].start()
    pcp[1].start()
    for h in range(2):
        pcp[h].wait()
        ocp[h].start()
    for h in range(2):
        ocp[h].wait()

    NC_CTX = len(CTX_LENS)
    NCHUNK = NC_CTX + len(SUF_LENS)

    def chunk_info(ci):
        if ci < NC_CTX:
            locb = CHUNK * ci

            def build(s):
                for n in range(7):
                    u = jnp.minimum(v_ctx + locb + 16 * n + j16, CTX_ROWS - 1)
                    cc = u - k_ctx * N_CLS
                    adj = jnp.where(cc >= N_CLS, 1, 0)
                    idx[s][pl.ds(16 * n, 16)] = k_ctx + adj

            return ctx_hbm, build, CTX_ROW0 + v_ctx + locb, CTX_LENS[ci]
        locb = CHUNK * (ci - NC_CTX)

        def build(s):
            for n in range(7):
                g = jnp.minimum(v_suf + locb + 16 * n + j16, SUF_ROWS - 1)
                cc = g - t_suf * N_CLS
                adj = jnp.where(cc >= N_CLS, 1, 0) + jnp.where(cc >= 2 * N_CLS, 1, 0)
                idx[s][pl.ds(16 * n, 16)] = (cc - adj * N_CLS) * SUF + (t_suf + adj)

        return suf_hbm, build, SUF_ROW0 + v_suf + locb, SUF_LENS[ci - NC_CTX]

    def g_copy(ci, s):
        table, _, _, _ = chunk_info(ci)
        return pltpu.make_async_copy(table.at[idx[s]], rows[s], gsem[s])

    def o_copy(ci, s):
        _, _, base, ln = chunk_info(ci)
        return pltpu.make_async_copy(
            rows[s].at[pl.ds(0, ln)], out_hbm.at[pl.ds(base, ln)], osem[s])

    def start_g(ci, s):
        _, build, _, _ = chunk_info(ci)
        build(s)
        g_copy(ci, s).start()

    start_g(0, 0)
    start_g(1, 1)
    for ci in range(NCHUNK):
        s = ci % 2
        g_copy(ci, s).wait()
        o_copy(ci, s).start()
        if ci + 2 < NCHUNK:
            o_copy(ci, s).wait()
            start_g(ci + 2, s)
    for ci in (NCHUNK - 2, NCHUNK - 1):
        o_copy(ci, ci % 2).wait()


def kernel(ctx, token_prefix, token_suffix):
    pre2d = jnp.transpose(token_prefix, (1, 0, 2)).reshape(PRE * N_CLS, D)
    suf2d = token_suffix.reshape(N_CLS * SUF, D)
    out2d = _assemble(ctx, pre2d, suf2d)
    out_t = out2d.reshape(TOT, N_CLS, D)
    return jnp.transpose(out_t, (1, 0, 2))


# speedup vs baseline: 2.0914x; 2.0914x over previous
import functools

import jax
import jax.numpy as jnp
from jax import lax
from jax.experimental import pallas as pl
from jax.experimental.pallas import tpu as pltpu
from jax.experimental.pallas import tpu_sc as plsc

N_CLS = 1000
PRE = 5
NCTX = 16
TOT = 77
SUF = TOT - PRE - NCTX
D = 512
NW = 32
CHUNK = 112

PRE_ROWS = PRE * N_CLS
CTX_ROWS = NCTX * N_CLS
SUF_ROWS = SUF * N_CLS
CTX_ROW0 = PRE_ROWS
SUF_ROW0 = PRE_ROWS + CTX_ROWS

SUF_PER_W = 1752
SUF_LENS = [CHUNK] * 15 + [SUF_PER_W - 15 * CHUNK]
NCHUNK = len(SUF_LENS)

_mesh = plsc.VectorSubcoreMesh(core_axis_name="c", subcore_axis_name="s")


@functools.partial(
    pl.kernel,
    mesh=_mesh,
    out_type=jax.ShapeDtypeStruct((TOT * N_CLS, D), jnp.float32),
    scratch_types=[
        pltpu.VMEM((CHUNK,), jnp.int32),
        pltpu.VMEM((CHUNK,), jnp.int32),
        pltpu.VMEM((16,), jnp.int32),
        pltpu.VMEM((CHUNK, D), jnp.float32),
        pltpu.VMEM((CHUNK, D), jnp.float32),
        pltpu.VMEM((16, D), jnp.float32),
        pltpu.SemaphoreType.DMA,
        pltpu.SemaphoreType.DMA,
        pltpu.SemaphoreType.DMA,
        pltpu.SemaphoreType.DMA,
        pltpu.SemaphoreType.DMA,
    ],
)
def _assemble(ctx_hbm, pre_hbm, suf_hbm, out_hbm,
              ia, ib, ic, ra, rb, crep, ga, gb, oa, ob, oc):
    idx = [ia, ib]
    rows = [ra, rb]
    gsem = [ga, gb]
    osem = [oa, ob]
    wid = lax.axis_index("s") * 2 + lax.axis_index("c")
    j16 = lax.iota(jnp.int32, 16)

    v_suf = jnp.minimum(wid * SUF_PER_W, SUF_ROWS - SUF_PER_W)
    t_suf = jnp.where(v_suf >= N_CLS, 1, 0)
    for r in range(2, SUF):
        t_suf = t_suf + jnp.where(v_suf >= r * N_CLS, 1, 0)

    a = jnp.minimum(wid * 160, PRE_ROWS - 160)
    pcp = [
        pltpu.make_async_copy(
            pre_hbm.at[pl.ds(a + 80 * h, 80)], rows[h].at[pl.ds(0, 80)],
            gsem[h])
        for h in range(2)
    ]
    ocp = [
        pltpu.make_async_copy(
            rows[h].at[pl.ds(0, 80)], out_hbm.at[pl.ds(a + 80 * h, 80)],
            osem[h])
        for h in range(2)
    ]
    pcp[0].start()
    pcp[1].start()
    for h in range(2):
        pcp[h].wait()
        ocp[h].start()

    k = wid // 2
    ic[pl.ds(0, 16)] = k + j16 * 0
    cg = pltpu.make_async_copy(ctx_hbm.at[ic], crep, oc)
    cg.start()
    cg.wait()
    cbase = CTX_ROW0 + k * N_CLS + jnp.minimum((wid % 2) * 504, N_CLS - 504)
    ccp = [
        pltpu.make_async_copy(
            crep, out_hbm.at[pl.ds(cbase + 16 * u, 16)], oc)
        for u in range(31)
    ] + [
        pltpu.make_async_copy(
            crep.at[pl.ds(0, 8)],
            out_hbm.at[pl.ds(cbase + 16 * 31, 8)], oc)
    ]
    for cp in ccp:
        cp.start()
    for h in range(2):
        ocp[h].wait()
    for cp in ccp:
        cp.wait()

    def build_idx(s, ci):
        locb = CHUNK * ci
        for n in range(7):
            g = jnp.minimum(v_suf + locb + 16 * n + j16, SUF_ROWS - 1)
            cc = g - t_suf * N_CLS
            adj = jnp.where(cc >= N_CLS, 1, 0) + jnp.where(cc >= 2 * N_CLS, 1, 0)
            idx[s][pl.ds(16 * n, 16)] = (cc - adj * N_CLS) * SUF + (t_suf + adj)

    def g_copy(s):
        return pltpu.make_async_copy(suf_hbm.at[idx[s]], rows[s], gsem[s])

    def o_copy(ci, s):
        ln = SUF_LENS[ci]
        return pltpu.make_async_copy(
            rows[s].at[pl.ds(0, ln)],
            out_hbm.at[pl.ds(SUF_ROW0 + v_suf + CHUNK * ci, ln)], osem[s])

    def start_g(ci, s):
        build_idx(s, ci)
        g_copy(s).start()

    start_g(0, 0)
    start_g(1, 1)
    for ci in range(NCHUNK):
        s = ci % 2
        g_copy(s).wait()
        o_copy(ci, s).start()
        if ci + 2 < NCHUNK:
            o_copy(ci, s).wait()
            start_g(ci + 2, s)
    for ci in (NCHUNK - 2, NCHUNK - 1):
        o_copy(ci, ci % 2).wait()


def kernel(ctx, token_prefix, token_suffix):
    pre2d = jnp.transpose(token_prefix, (1, 0, 2)).reshape(PRE * N_CLS, D)
    suf2d = token_suffix.reshape(N_CLS * SUF, D)
    out2d = _assemble(ctx, pre2d, suf2d)
    out_t = out2d.reshape(TOT, N_CLS, D)
    return jnp.transpose(out_t, (1, 0, 2))


# speedup vs baseline: 2.1055x; 1.0067x over previous
import functools

import jax
import jax.numpy as jnp
from jax import lax
from jax.experimental import pallas as pl
from jax.experimental.pallas import tpu as pltpu
from jax.experimental.pallas import tpu_sc as plsc

N_CLS = 1000
PRE = 5
NCTX = 16
TOT = 77
SUF = TOT - PRE - NCTX
D = 512
NW = 32
CHUNK = 112
NCHUNK = 9
TAIL = N_CLS - 8 * CHUNK

PRE_ROWS = PRE * N_CLS
CTX_ROW0 = PRE_ROWS
SUF_ROW0 = (PRE + NCTX) * N_CLS

_mesh = plsc.VectorSubcoreMesh(core_axis_name="c", subcore_axis_name="s")


@functools.partial(
    pl.kernel,
    mesh=_mesh,
    out_type=jax.ShapeDtypeStruct((TOT * N_CLS, D), jnp.float32),
    scratch_types=[
        pltpu.VMEM((CHUNK,), jnp.int32),
        pltpu.VMEM((CHUNK,), jnp.int32),
        pltpu.VMEM((16,), jnp.int32),
        pltpu.VMEM((CHUNK, D), jnp.float32),
        pltpu.VMEM((CHUNK, D), jnp.float32),
        pltpu.VMEM((16, D), jnp.float32),
        pltpu.SemaphoreType.DMA,
        pltpu.SemaphoreType.DMA,
        pltpu.SemaphoreType.DMA,
        pltpu.SemaphoreType.DMA,
        pltpu.SemaphoreType.DMA,
    ],
)
def _assemble(ctx_hbm, pre_hbm, suf_hbm, out_hbm,
              ia, ib, ic, ra, rb, crep, ga, gb, oa, ob, oc):
    idx = [ia, ib]
    rows = [ra, rb]
    gsem = [ga, gb]
    osem = [oa, ob]
    wid = lax.axis_index("s") * 2 + lax.axis_index("c")
    j16 = lax.iota(jnp.int32, 16)

    a = jnp.minimum(wid * 160, PRE_ROWS - 160)
    pcp = [
        pltpu.make_async_copy(
            pre_hbm.at[pl.ds(a + 80 * h, 80)], rows[h].at[pl.ds(0, 80)],
            gsem[h])
        for h in range(2)
    ]
    ocp = [
        pltpu.make_async_copy(
            rows[h].at[pl.ds(0, 80)], out_hbm.at[pl.ds(a + 80 * h, 80)],
            osem[h])
        for h in range(2)
    ]
    pcp[0].start()
    pcp[1].start()
    for h in range(2):
        pcp[h].wait()
        ocp[h].start()

    k = wid // 2
    ic[pl.ds(0, 16)] = k + j16 * 0
    cg = pltpu.make_async_copy(ctx_hbm.at[ic], crep, oc)
    cg.start()
    cg.wait()
    cbase = CTX_ROW0 + k * N_CLS + jnp.minimum((wid % 2) * 504, N_CLS - 504)
    ccp = [
        pltpu.make_async_copy(
            crep, out_hbm.at[pl.ds(cbase + 16 * u, 16)], oc)
        for u in range(31)
    ] + [
        pltpu.make_async_copy(
            crep.at[pl.ds(0, 8)],
            out_hbm.at[pl.ds(cbase + 16 * 31, 8)], oc)
    ]
    for cp in ccp:
        cp.start()
    for h in range(2):
        ocp[h].wait()
    for cp in ccp:
        cp.wait()

    def build_sidx(s, ci, t):
        c0 = CHUNK * ci
        for n in range(7):
            c = jnp.minimum(c0 + 16 * n + j16, N_CLS - 1)
            idx[s][pl.ds(16 * n, 16)] = c * SUF + t

    def g_copy(s):
        return pltpu.make_async_copy(suf_hbm.at[idx[s]], rows[s], gsem[s])

    def o_copy(s, ci, t):
        c0 = CHUNK * ci
        ln = CHUNK if ci < NCHUNK - 1 else TAIL
        return pltpu.make_async_copy(
            rows[s].at[pl.ds(0, ln)],
            out_hbm.at[pl.ds(SUF_ROW0 + t * N_CLS + c0, ln)], osem[s])

    def do_plane(t):
        build_sidx(0, 0, t)
        g_copy(0).start()
        build_sidx(1, 1, t)
        g_copy(1).start()
        for ci in range(NCHUNK):
            s = ci % 2
            g_copy(s).wait()
            o_copy(s, ci, t).start()
            if ci + 2 < NCHUNK:
                o_copy(s, ci, t).wait()
                build_sidx(s, ci + 2, t)
                g_copy(s).start()
        for ci in (NCHUNK - 2, NCHUNK - 1):
            o_copy(ci % 2, ci, t).wait()

    do_plane(wid)

    @pl.when(wid < SUF - 32)
    def _():
        do_plane(wid + 32)


def kernel(ctx, token_prefix, token_suffix):
    pre2d = jnp.transpose(token_prefix, (1, 0, 2)).reshape(PRE * N_CLS, D)
    suf2d = token_suffix.reshape(N_CLS * SUF, D)
    out2d = _assemble(ctx, pre2d, suf2d)
    out_t = out2d.reshape(TOT, N_CLS, D)
    return jnp.transpose(out_t, (1, 0, 2))
